# Initial kernel scaffold; baseline (speedup 1.0000x reference)
import jax
import jax.numpy as jnp
from jax.experimental import pallas as pl


def kernel(hidden_states, Wg, gate_w, up_w, down_w):
    raise NotImplementedError("write your pallas kernel here")


# speedup vs baseline: 1.7079x; 1.7079x over previous
import functools

import jax
import jax.numpy as jnp
from jax.experimental import pallas as pl

_NUM_EXPERTS = 8


def _router_kernel(x_ref, wg_ref, w_ref):
    x = x_ref[...]
    logits = jax.lax.dot_general(
        x, wg_ref[...], (((1,), (1,)), ((), ())),
        preferred_element_type=jnp.float32)
    m = jnp.max(logits, axis=-1, keepdims=True)
    p = jnp.exp(logits - m)
    p = p / jnp.sum(p, axis=-1, keepdims=True)
    n, e = p.shape
    lane = jax.lax.broadcasted_iota(jnp.int32, (n, e), 1)
    i1 = jnp.argmax(p, axis=-1, keepdims=True)
    v1 = jnp.max(p, axis=-1, keepdims=True)
    oh1 = (lane == i1).astype(p.dtype)
    p2 = p * (1.0 - oh1)
    i2 = jnp.argmax(p2, axis=-1, keepdims=True)
    v2 = jnp.max(p2, axis=-1, keepdims=True)
    oh2 = (lane == i2).astype(p.dtype)
    w_ref[...] = (oh1 * v1 + oh2 * v2) / (v1 + v2)


def _moe_kernel(w_ref, x_ref, gw_ref, uw_ref, dw_ref, out_ref):
    e = pl.program_id(0)
    x = x_ref[...]
    g = jax.lax.dot_general(
        x, gw_ref[0], (((1,), (1,)), ((), ())),
        preferred_element_type=jnp.float32)
    u = jax.lax.dot_general(
        x, uw_ref[0], (((1,), (1,)), ((), ())),
        preferred_element_type=jnp.float32)
    h = (g * jax.nn.sigmoid(g) * u).astype(jnp.bfloat16)
    y = jax.lax.dot_general(
        h, dw_ref[0], (((1,), (1,)), ((), ())),
        preferred_element_type=jnp.float32)
    w = w_ref[...]
    n, ne = w.shape
    lane = jax.lax.broadcasted_iota(jnp.int32, (n, ne), 1)
    wcol = jnp.sum(jnp.where(lane == e, w, 0.0), axis=1, keepdims=True)
    contrib = wcol * y

    @pl.when(e == 0)
    def _():
        out_ref[...] = contrib

    @pl.when(e > 0)
    def _():
        out_ref[...] += contrib


@functools.partial(jax.jit, static_argnames=())
def kernel(hidden_states, Wg, gate_w, up_w, down_w):
    B, S, H = hidden_states.shape
    E, I, _ = gate_w.shape
    N = B * S
    x = hidden_states.reshape(N, H)

    w = pl.pallas_call(
        _router_kernel,
        grid=(1,),
        in_specs=[
            pl.BlockSpec((N, H), lambda i: (0, 0)),
            pl.BlockSpec((E, H), lambda i: (0, 0)),
        ],
        out_specs=pl.BlockSpec((N, E), lambda i: (0, 0)),
        out_shape=jax.ShapeDtypeStruct((N, E), jnp.float32),
    )(x, Wg)

    x16 = x.astype(jnp.bfloat16)
    gw16 = gate_w.astype(jnp.bfloat16)
    uw16 = up_w.astype(jnp.bfloat16)
    dw16 = down_w.astype(jnp.bfloat16)

    out = pl.pallas_call(
        _moe_kernel,
        grid=(E,),
        in_specs=[
            pl.BlockSpec((N, E), lambda e: (0, 0)),
            pl.BlockSpec((N, H), lambda e: (0, 0)),
            pl.BlockSpec((1, I, H), lambda e: (e, 0, 0)),
            pl.BlockSpec((1, I, H), lambda e: (e, 0, 0)),
            pl.BlockSpec((1, H, I), lambda e: (e, 0, 0)),
        ],
        out_specs=pl.BlockSpec((N, H), lambda e: (0, 0)),
        out_shape=jax.ShapeDtypeStruct((N, H), jnp.float32),
    )(w, x16, gw16, uw16, dw16)

    return out.reshape(B, S, H)


# speedup vs baseline: 2.3685x; 1.3868x over previous
import functools

import jax
import jax.numpy as jnp
from jax.experimental import pallas as pl

_NUM_EXPERTS = 8


def _router_kernel(x_ref, wg_ref, w_ref, x16_ref):
    x = x_ref[...]
    logits = jax.lax.dot_general(
        x, wg_ref[...], (((1,), (1,)), ((), ())),
        preferred_element_type=jnp.float32)
    m = jnp.max(logits, axis=-1, keepdims=True)
    p = jnp.exp(logits - m)
    p = p / jnp.sum(p, axis=-1, keepdims=True)
    n, e = p.shape
    lane = jax.lax.broadcasted_iota(jnp.int32, (n, e), 1)
    i1 = jnp.argmax(p, axis=-1, keepdims=True)
    v1 = jnp.max(p, axis=-1, keepdims=True)
    oh1 = (lane == i1).astype(p.dtype)
    p2 = p * (1.0 - oh1)
    i2 = jnp.argmax(p2, axis=-1, keepdims=True)
    v2 = jnp.max(p2, axis=-1, keepdims=True)
    oh2 = (lane == i2).astype(p.dtype)
    w_ref[...] = (oh1 * v1 + oh2 * v2) / (v1 + v2)
    x16_ref[...] = x.astype(jnp.bfloat16)


def _moe_kernel(w_ref, x_ref, gw_ref, uw_ref, dw_ref, out_ref):
    e = pl.program_id(0)
    x = x_ref[...]
    g = jax.lax.dot_general(
        x, gw_ref[0].astype(jnp.bfloat16), (((1,), (1,)), ((), ())),
        preferred_element_type=jnp.float32)
    u = jax.lax.dot_general(
        x, uw_ref[0].astype(jnp.bfloat16), (((1,), (1,)), ((), ())),
        preferred_element_type=jnp.float32)
    h = (g * jax.nn.sigmoid(g) * u).astype(jnp.bfloat16)
    y = jax.lax.dot_general(
        h, dw_ref[0].astype(jnp.bfloat16), (((1,), (1,)), ((), ())),
        preferred_element_type=jnp.float32)
    w = w_ref[...]
    n, ne = w.shape
    lane = jax.lax.broadcasted_iota(jnp.int32, (n, ne), 1)
    wcol = jnp.sum(jnp.where(lane == e, w, 0.0), axis=1, keepdims=True)
    contrib = wcol * y

    @pl.when(e == 0)
    def _():
        out_ref[...] = contrib

    @pl.when(e > 0)
    def _():
        out_ref[...] += contrib


@functools.partial(jax.jit, static_argnames=())
def kernel(hidden_states, Wg, gate_w, up_w, down_w):
    B, S, H = hidden_states.shape
    E, I, _ = gate_w.shape
    N = B * S
    x = hidden_states.reshape(N, H)

    w, x16 = pl.pallas_call(
        _router_kernel,
        grid=(1,),
        in_specs=[
            pl.BlockSpec((N, H), lambda i: (0, 0)),
            pl.BlockSpec((E, H), lambda i: (0, 0)),
        ],
        out_specs=[
            pl.BlockSpec((N, E), lambda i: (0, 0)),
            pl.BlockSpec((N, H), lambda i: (0, 0)),
        ],
        out_shape=[
            jax.ShapeDtypeStruct((N, E), jnp.float32),
            jax.ShapeDtypeStruct((N, H), jnp.bfloat16),
        ],
    )(x, Wg)

    out = pl.pallas_call(
        _moe_kernel,
        grid=(E,),
        in_specs=[
            pl.BlockSpec((N, E), lambda e: (0, 0)),
            pl.BlockSpec((N, H), lambda e: (0, 0)),
            pl.BlockSpec((1, I, H), lambda e: (e, 0, 0)),
            pl.BlockSpec((1, I, H), lambda e: (e, 0, 0)),
            pl.BlockSpec((1, H, I), lambda e: (e, 0, 0)),
        ],
        out_specs=pl.BlockSpec((N, H), lambda e: (0, 0)),
        out_shape=jax.ShapeDtypeStruct((N, H), jnp.float32),
    )(w, x16, gate_w, up_w, down_w)

    return out.reshape(B, S, H)
